# baseline (speedup 1.0000x reference)
import functools

import jax
import jax.numpy as jnp
from jax import lax
from jax.experimental import pallas as pl
from jax.experimental.pallas import tpu as pltpu
from jax.experimental.pallas import tpu_sc as plsc

B = 1024
L = 50
D = 64
V = 1000000
NC = 2
NS = 16
NW = NC * NS
PPW = (B * L) // NW
GSZ = 16
NGRP = 104
PPAD = NGRP * GSZ
EW = 1024
VMAIN = 999936
TAILW = 64
MAXC0 = VMAIN - EW
ACCN = 1152
TRASHROW = 1024
ZROWS = 128


@functools.cache
def _get_mesh():
    return plsc.VectorSubcoreMesh(
        core_axis_name="c", subcore_axis_name="s", num_cores=NC, num_subcores=NS
    )


def _scan_body(sv_hbm, se_hbm, xt_hbm, out_hbm,
               sv_v, se_v, slab, tail, stage, tgt_v, zbuf, acc_sh,
               sem1, sem2, sem3):
    sid = lax.axis_index("s")
    cid = lax.axis_index("c")
    wid = sid * NC + cid
    base = 0
    lanes = lax.broadcasted_iota(jnp.int32, (16,), 0)

    c1 = pltpu.async_copy(sv_hbm.at[wid], sv_v.at[...], sem1)
    c2 = pltpu.async_copy(se_hbm.at[wid], se_v.at[...], sem2)
    c3 = pltpu.async_copy(xt_hbm.at[:, pl.ds(VMAIN, TAILW)], tail.at[...], sem3)

    zeros16 = jnp.zeros((16,), jnp.float32)

    @pl.when(sid == 0)
    def _():
        def zrow(r, carry):
            for j in range(4):
                zbuf[r, pl.ds(j * 16, 16)] = zeros16
            return carry

        lax.fori_loop(0, ZROWS, zrow, 0)
        for j in range(ACCN // ZROWS):
            pltpu.sync_copy(
                zbuf.at[...], acc_sh.at[pl.ds(j * ZROWS, ZROWS)]
            )

    plsc.subcore_barrier()
    c1.wait()
    c2.wait()
    c3.wait()

    def group(g, c0):
        sv16 = sv_v[g >> 3, pl.ds((g & 7) * GSZ, GSZ)]
        se16 = se_v[g >> 3, pl.ds((g & 7) * GSZ, GSZ)]

        tmask = sv16 >= VMAIN
        tcols = jnp.clip(sv16 - VMAIN, 0, TAILW - 1)
        remaining = sv16 < VMAIN

        def wcond(carry):
            c0_, rem, tm = carry
            return jnp.any(rem) | jnp.any(tm)

        def wbody(carry):
            c0_, rem, tm = carry
            mv = jnp.min(jnp.where(rem, sv16, jnp.int32(2**30)))
            need = jnp.any(rem) & (mv >= c0_ + EW)
            c0n = jnp.where(
                need, jnp.minimum(mv & ~jnp.int32(127), jnp.int32(MAXC0)), c0_
            )
            c0n = pl.multiple_of(c0n, 128)

            @pl.when(need)
            def _():
                pltpu.sync_copy(
                    xt_hbm.at[:, pl.ds(c0n, EW)], slab.at[...]
                )

            inw = rem & (sv16 >= c0n) & (sv16 < c0n + EW)
            cols = jnp.clip(sv16 - c0n, 0, EW - 1)
            for f in range(D):
                    frow = jnp.full((16,), f, jnp.int32)
                    gv = plsc.load_gather(slab, [frow, cols], mask=inw)
                    tv = plsc.load_gather(tail, [frow, tcols], mask=tm)
                    plsc.store_scatter(
                        stage, [lanes, jnp.full((16,), f, jnp.int32)],
                        gv, mask=inw,
                    )
                    plsc.store_scatter(
                        stage, [lanes, jnp.full((16,), f, jnp.int32)],
                        tv, mask=tm,
                    )
            done = inw | tm
            tgt = jnp.where(done, se16 + base, jnp.int32(base + TRASHROW))
            tgt_v[g & 1, pl.ds(0, 16)] = tgt
            pltpu.sync_copy(
                stage.at[...], acc_sh.at[tgt_v.at[g & 1]], add=True
            )
            return c0n, rem & ~inw, tm & ~tm

        c0, _, _ = lax.while_loop(wcond, wbody, (c0, remaining, tmask))
        return c0

    lax.fori_loop(0, NGRP, group, jnp.int32(-2 * EW))

    plsc.subcore_barrier()

    @pl.when(sid == 0)
    def _():
        pltpu.sync_copy(acc_sh.at[pl.ds(0, B)], out_hbm.at[cid])


@functools.cache
def _get_scan():
    return pl.kernel(
        _scan_body,
        out_type=jax.ShapeDtypeStruct((NC, B, D), jnp.float32),
        mesh=_get_mesh(),
        scratch_types=[
            pltpu.VMEM((PPAD // 128, 128), jnp.int32),
            pltpu.VMEM((PPAD // 128, 128), jnp.int32),
            pltpu.VMEM((D, EW), jnp.float32),
            pltpu.VMEM((D, TAILW), jnp.float32),
            pltpu.VMEM((GSZ, D), jnp.float32),
            pltpu.VMEM((2, 16), jnp.int32),
            pltpu.VMEM((ZROWS, D), jnp.float32),
            pltpu.VMEM_SHARED((ACCN, D), jnp.float32),
            pltpu.SemaphoreType.DMA,
            pltpu.SemaphoreType.DMA,
            pltpu.SemaphoreType.DMA,
        ],
        compiler_params=pltpu.CompilerParams(
            use_tc_tiling_on_sc=False, needs_layout_passes=False
        ),
    )


def _linear_body(acc_ref, wt_ref, b_ref, out_ref):
    a = acc_ref[...]
    his = jnp.sum(a, axis=0)
    out_ref[...] = (
        jnp.dot(his, wt_ref[...], preferred_element_type=jnp.float32)
        + b_ref[...]
    )


def kernel(entities, history, entities_emb, W, b):
    del entities
    flat = history.astype(jnp.int32).reshape(B * L)
    ev = (jnp.arange(B * L, dtype=jnp.int32) // L)
    packed = jnp.sort(flat * jnp.int32(2048) + ev)
    sv = packed >> 11
    se = packed & jnp.int32(2047)
    sv = sv.reshape(NW, PPW)
    se = se.reshape(NW, PPW)
    padv = jnp.broadcast_to(sv[:, -1:], (NW, PPAD - PPW))
    sv = jnp.concatenate([sv, padv], axis=1).reshape(NW, PPAD // 128, 128)
    se = jnp.pad(
        se, ((0, 0), (0, PPAD - PPW)), constant_values=TRASHROW
    ).reshape(NW, PPAD // 128, 128)

    acc = _get_scan()(sv, se, entities_emb.T)

    out = pl.pallas_call(
        _linear_body,
        out_shape=jax.ShapeDtypeStruct((B, D), jnp.float32),
    )(acc, W.T, b.reshape(1, D))
    return out
